# baseline (speedup 1.0000x reference)
import functools

import jax
import jax.numpy as jnp
from jax import lax
from jax.experimental import pallas as pl
from jax.experimental.pallas import tpu as pltpu
from jax.experimental.pallas import tpu_sc as plsc

DIM = 64
BLOCK_LEN = 4
NUM_BLOCKS = DIM // BLOCK_LEN
LANES = 16
K = 1024

NUM_CORES = 2
NUM_SUBCORES = 16
NW = NUM_CORES * NUM_SUBCORES

CHUNK = 400


@functools.partial(jax.jit, static_argnames=("n_chunks", "bdim", "ldim"))
def _decode(idx_flat, codes, cb_flat, n_chunks, bdim, ldim):
    n = idx_flat.shape[0]
    mesh = plsc.VectorSubcoreMesh(core_axis_name="c", subcore_axis_name="s")

    @functools.partial(
        pl.kernel,
        out_type=jax.ShapeDtypeStruct((bdim, ldim, DIM), jnp.float32),
        mesh=mesh,
        compiler_params=pltpu.CompilerParams(
            use_tc_tiling_on_sc=False, needs_layout_passes=False
        ),
        scratch_types=[
            pltpu.VMEM((K * BLOCK_LEN,), jnp.float32),
            [pltpu.VMEM((CHUNK,), jnp.int32)] * 2,
            [pltpu.VMEM((CHUNK, NUM_BLOCKS), jnp.int32)] * 2,
            [pltpu.VMEM((CHUNK // ldim, ldim, DIM), jnp.float32)] * 2,
            [pltpu.SemaphoreType.DMA] * 2,
            [pltpu.SemaphoreType.DMA] * 2,
        ],
    )
    def k(idx_hbm, codes_hbm, cb_hbm, out_hbm, cb_v, idx_v, codes_v, dense_v,
          gsem, osem):
        rpc = CHUNK // ldim
        wid = lax.axis_index("s") * NUM_CORES + lax.axis_index("c")
        per_w = n // NW
        base0 = wid * per_w
        row0 = wid * (per_w // ldim)
        lanes = lax.iota(jnp.int32, LANES)
        sel = lanes >> 2
        pat = lanes & 3
        pltpu.sync_copy(cb_hbm, cb_v)

        def fetch(ci, b):
            pltpu.sync_copy(
                idx_hbm.at[pl.ds(base0 + ci * CHUNK, CHUNK)], idx_v[b]
            )
            pltpu.async_copy(codes_hbm.at[idx_v[b]], codes_v[b], gsem[b])

        fetch(0, 0)

        @pl.loop(0, n_chunks, step=2)
        def outer(ci0):
            for b in (0, 1):
                ci = ci0 + b

                @pl.when(ci + 1 < n_chunks)
                def _():
                    fetch(ci + 1, 1 - b)

                pltpu.make_async_copy(
                    codes_hbm.at[idx_v[b]], codes_v[b], gsem[b]
                ).wait()

                @pl.when(ci >= 2)
                def _():
                    pltpu.make_async_copy(
                        dense_v[b],
                        out_hbm.at[pl.ds(row0, rpc)],
                        osem[b],
                    ).wait()

                cvb = codes_v[b]
                dvb = dense_v[b].reshape(CHUNK, DIM)

                @functools.partial(plsc.parallel_loop, 0, CHUNK, unroll=2)
                def decode(i):
                    row = jnp.full((LANES,), i, dtype=jnp.int32)
                    for v in range(DIM // LANES):
                        rid = plsc.load_gather(
                            cvb, [row, sel + v * BLOCK_LEN]
                        )
                        fid = (rid << 2) + pat
                        vals = plsc.load_gather(cb_v, [fid])
                        dvb[i, pl.ds(v * LANES, LANES)] = vals

                pltpu.async_copy(
                    dense_v[b],
                    out_hbm.at[pl.ds(row0 + ci * rpc, rpc)],
                    osem[b],
                )

        for b in (0, 1):
            pltpu.make_async_copy(
                dense_v[b],
                out_hbm.at[pl.ds(row0, rpc)],
                osem[b],
            ).wait()

    return k(idx_flat, codes, cb_flat)


def kernel(indices, codes, codebook):
    b, l = indices.shape
    n = b * l
    assert n % (NW * CHUNK) == 0 and (n // (NW * CHUNK)) % 2 == 0
    assert CHUNK % l == 0 and (n // NW) % l == 0
    return _decode(
        indices.reshape(n), codes, codebook.reshape(K * BLOCK_LEN),
        n_chunks=n // (NW * CHUNK), bdim=b, ldim=l,
    )


# speedup vs baseline: 2.4384x; 2.4384x over previous
import functools

import jax
import jax.numpy as jnp
from jax import lax
from jax.experimental import pallas as pl
from jax.experimental.pallas import tpu as pltpu
from jax.experimental.pallas import tpu_sc as plsc

DIM = 64
BLOCK_LEN = 4
NUM_BLOCKS = DIM // BLOCK_LEN
LANES = 16
K = 1024

NUM_CORES = 2
NUM_SUBCORES = 16
NW = NUM_CORES * NUM_SUBCORES

BW = 128
L5 = 5


@functools.partial(jax.jit, static_argnames=("bdim", "ldim"))
def _decode(idx_flat, codes, cb_flat, bdim, ldim):
    n = bdim * ldim
    n_chunks = ldim // L5
    per_w = n // NW
    mesh = plsc.VectorSubcoreMesh(core_axis_name="c", subcore_axis_name="s")

    @functools.partial(
        pl.kernel,
        out_type=jax.ShapeDtypeStruct(
            (ldim, 8, bdim // BW, 8, BW), jnp.float32
        ),
        mesh=mesh,
        compiler_params=pltpu.CompilerParams(
            use_tc_tiling_on_sc=False, needs_layout_passes=False
        ),
        scratch_types=[
            pltpu.VMEM((K * BLOCK_LEN,), jnp.float32),
            pltpu.VMEM((per_w,), jnp.int32),
            [pltpu.VMEM((L5 * BW,), jnp.int32)] * 2,
            [pltpu.VMEM((L5 * BW, NUM_BLOCKS), jnp.int32)] * 2,
            [pltpu.VMEM((L5, 8, 8, BW), jnp.float32)] * 2,
            [pltpu.SemaphoreType.DMA] * 2,
            [pltpu.SemaphoreType.DMA] * 2,
        ],
    )
    def k(idx_hbm, codes_hbm, cb_hbm, out_hbm, cb_v, idx_v, glist, codes_v,
          dense_v, gsem, osem):
        wid = lax.axis_index("s") * NUM_CORES + lax.axis_index("c")
        base0 = wid * per_w
        lanes = lax.iota(jnp.int32, LANES)
        lanes_l = lanes * ldim
        pltpu.sync_copy(cb_hbm, cb_v)
        pltpu.sync_copy(idx_hbm.at[pl.ds(base0, per_w)], idx_v)

        def build_and_fetch(ci, b):
            @pl.loop(0, L5)
            def _l(ll):
                @pl.loop(0, BW // LANES)
                def _g(bg):
                    pos = lanes_l + (bg * LANES * ldim + ci * L5 + ll)
                    vals = plsc.load_gather(idx_v, [pos])
                    glist[b][pl.ds(ll * BW + bg * LANES, LANES)] = vals

            plsc.subcore_barrier()
            pltpu.async_copy(codes_hbm.at[glist[b]], codes_v[b], gsem[b])

        build_and_fetch(0, 0)

        @pl.loop(0, n_chunks, step=2)
        def outer(ci0):
            for b in (0, 1):
                ci = ci0 + b

                @pl.when(ci + 1 < n_chunks)
                def _():
                    build_and_fetch(ci + 1, 1 - b)

                pltpu.make_async_copy(
                    codes_hbm.at[glist[b]], codes_v[b], gsem[b]
                ).wait()

                @pl.when(ci >= 2)
                def _():
                    for ll in range(L5):
                        for dh in range(8):
                            pltpu.make_async_copy(
                                dense_v[b].at[ll, dh],
                                out_hbm.at[ll, dh, wid],
                                osem[b],
                            ).wait()

                cvb = codes_v[b]
                dvb = dense_v[b].reshape(L5 * DIM, BW)

                @pl.loop(0, L5)
                def dec_l(ll):
                    @functools.partial(
                        plsc.parallel_loop, 0, BW // LANES, unroll=2
                    )
                    def dec_g(bg):
                        row = jnp.full(
                            (LANES,), ll * BW + bg * LANES, dtype=jnp.int32
                        ) + lanes
                        for dg in range(NUM_BLOCKS):
                            col = jnp.full((LANES,), dg, dtype=jnp.int32)
                            rid = plsc.load_gather(cvb, [row, col])
                            fid = rid << 2
                            for dlo in range(BLOCK_LEN):
                                vals = plsc.load_gather(cb_v, [fid + dlo])
                                d = dg * BLOCK_LEN + dlo
                                dvb[ll * DIM + d,
                                    pl.ds(bg * LANES, LANES)] = vals

                for ll in range(L5):
                    for dh in range(8):
                        pltpu.async_copy(
                            dense_v[b].at[ll, dh],
                            out_hbm.at[ci * L5 + ll, dh, wid],
                            osem[b],
                        )

        for b in (0, 1):
            for ll in range(L5):
                for dh in range(8):
                    pltpu.make_async_copy(
                        dense_v[b].at[ll, dh],
                        out_hbm.at[ll, dh, wid],
                        osem[b],
                    ).wait()

    return k(idx_flat, codes, cb_flat)


def kernel(indices, codes, codebook):
    b, l = indices.shape
    n = b * l
    assert b % (NW * BW) == 0 or b % BW == 0
    assert b // BW == NW
    assert l % (2 * L5) == 0
    out5 = _decode(
        indices.reshape(n), codes, codebook.reshape(K * BLOCK_LEN),
        bdim=b, ldim=l,
    )
    return out5.transpose(2, 4, 0, 1, 3).reshape(b, l, DIM)
